# Initial kernel scaffold; baseline (speedup 1.0000x reference)
import jax
import jax.numpy as jnp
from jax.experimental import pallas as pl


def kernel(x, rel0, W0, b0, g0, be0, rel1, W1, b1, g1, be1, rel2, W2, b2, g2, be2, edge_index, edge_type):
    raise NotImplementedError("write your pallas kernel here")


# speedup vs baseline: 2.6200x; 2.6200x over previous
import functools

import jax
import jax.numpy as jnp
from jax import lax
from jax.experimental import pallas as pl
from jax.experimental.pallas import tpu as pltpu
from jax.experimental.pallas import tpu_sc as plsc

N = 10000
E = 320000
D = 128
NR = 36
EPS = 1e-5

VT = 64
NPT = 160
NPAD = VT * NPT
CHUNK = 128
EPAD = E + CHUNK
BLK = 1024
FMAX = 3.4e38


def _edge_body(xh, srch, dsth, eth, bndh, relh,
               osum, osq, omx, omn,
               rel_v, bnd_v, idx_v, dst_v, et_v, rows_v,
               acc_s, acc_q, acc_m, acc_n, sem):
    cid = lax.axis_index("c")
    sid = lax.axis_index("s")
    wid = sid * 2 + cid
    pltpu.sync_copy(relh, rel_v)
    pltpu.sync_copy(bndh, bnd_v)
    zero = jnp.zeros((16,), jnp.float32)
    neg = jnp.full((16,), -FMAX, jnp.float32)
    pos = jnp.full((16,), FMAX, jnp.float32)

    for p in range(2):
        v = wid * 2 + p
        node_base = v * NPT
        bv = bnd_v[v, pl.ds(0, 16)]
        e_start = bv[0]
        e_end = bv[1]

        def init_row(i, _):
            for j in range(8):
                sl = pl.ds(j * 16, 16)
                acc_s[i, sl] = zero
                acc_q[i, sl] = zero
                acc_m[i, sl] = neg
                acc_n[i, sl] = pos
            return 0

        lax.fori_loop(0, NPT + 1, init_row, 0)

        c0 = (e_start // 8) * 8
        nch = (e_end - c0 + CHUNK - 1) // CHUNK

        def chunk(kc, _):
            base = c0 + kc * CHUNK
            pltpu.sync_copy(srch.at[pl.ds(base, CHUNK)], idx_v)
            pltpu.sync_copy(dsth.at[pl.ds(base, CHUNK)], dst_v.at[pl.ds(0, CHUNK)])
            pltpu.sync_copy(eth.at[pl.ds(base, CHUNK)], et_v.at[pl.ds(0, CHUNK)])
            pltpu.async_copy(xh.at[idx_v], rows_v, sem).wait()
            lo = jnp.maximum(e_start, base)
            hi = jnp.minimum(e_end, base + CHUNK)

            def edge(e, _):
                off = e - base
                ld = dst_v[pl.ds(off, 16)][0] - node_base
                etk = et_v[pl.ds(off, 16)][0]
                for j in range(8):
                    sl = pl.ds(j * 16, 16)
                    m = rel_v[etk, sl] * rows_v[off, sl]
                    plsc.addupdate(acc_s.at[ld, sl], m)
                    plsc.addupdate(acc_q.at[ld, sl], m * m)
                    acc_m[ld, sl] = jnp.maximum(acc_m[ld, sl], m)
                    acc_n[ld, sl] = jnp.minimum(acc_n[ld, sl], m)
                return 0

            lax.fori_loop(lo, hi, edge, 0)
            return 0

        lax.fori_loop(0, nch, chunk, 0)
        pltpu.sync_copy(acc_s.at[pl.ds(0, NPT)], osum.at[pl.ds(node_base, NPT)])
        pltpu.sync_copy(acc_q.at[pl.ds(0, NPT)], osq.at[pl.ds(node_base, NPT)])
        pltpu.sync_copy(acc_m.at[pl.ds(0, NPT)], omx.at[pl.ds(node_base, NPT)])
        pltpu.sync_copy(acc_n.at[pl.ds(0, NPT)], omn.at[pl.ds(node_base, NPT)])


_edge_call = pl.kernel(
    _edge_body,
    out_type=[jax.ShapeDtypeStruct((NPAD, D), jnp.float32)] * 4,
    mesh=plsc.VectorSubcoreMesh(core_axis_name="c", subcore_axis_name="s"),
    scratch_types=[
        pltpu.VMEM((NR, D), jnp.float32),
        pltpu.VMEM((VT, 16), jnp.int32),
        pltpu.VMEM((CHUNK,), jnp.int32),
        pltpu.VMEM((CHUNK + 16,), jnp.int32),
        pltpu.VMEM((CHUNK + 16,), jnp.int32),
        pltpu.VMEM((CHUNK, D), jnp.float32),
        pltpu.VMEM((NPT + 1, D), jnp.float32),
        pltpu.VMEM((NPT + 1, D), jnp.float32),
        pltpu.VMEM((NPT + 1, D), jnp.float32),
        pltpu.VMEM((NPT + 1, D), jnp.float32),
        pltpu.SemaphoreType.DMA,
    ],
)


def _mld_body(lo_ref, hi_ref, out_ref):
    degf = (hi_ref[...] - lo_ref[...] + 1).astype(jnp.float32)
    out_ref[...] = (jnp.sum(jnp.log(degf)) / jnp.float32(N)).reshape(1, 1)


def _node_body(h_ref, bnd_ref, s_ref, q_ref, m_ref, n_ref,
               lo_ref, hi_ref, mld_ref, wp_ref, b_ref, g_ref, be_ref, out_ref):
    x = bnd_ref[...]
    h = h_ref[...]
    deg = (hi_ref[...] - lo_ref[...] + 1).astype(jnp.float32)
    st = s_ref[...] + x
    qt = q_ref[...] + x * x
    mx = jnp.maximum(m_ref[...], x)
    mn = jnp.minimum(n_ref[...], x)
    inv_deg = 1.0 / deg
    mean = st * inv_deg
    std = jnp.sqrt(jnp.clip(qt * inv_deg - mean * mean, 1e-10, None))
    sc = jnp.log(deg) / mld_ref[0, 0]
    inv = 1.0 / jnp.clip(sc, 1e-2, None)
    parts = [h]
    for f in (mean, mx, mn, std):
        parts += [f, f * sc, f * inv]
    feats = jnp.concatenate(parts, axis=1)
    out = lax.dot_general(feats, wp_ref[...], (((1,), (0,)), ((), ())),
                          preferred_element_type=jnp.float32) + b_ref[...]
    mu = jnp.mean(out, axis=-1, keepdims=True)
    var = jnp.mean((out - mu) ** 2, axis=-1, keepdims=True)
    out = (out - mu) / jnp.sqrt(var + EPS) * g_ref[...] + be_ref[...]
    out_ref[...] = jnp.maximum(out, 0.0) + h


def _node_call(h, bnd, s, q, m, n, lo, hi, mld, wp, b, g, be):
    grid = NPAD // BLK
    row = pl.BlockSpec((BLK, D), lambda i: (i, 0))
    col = pl.BlockSpec((BLK, 1), lambda i: (i, 0))
    full = lambda shape: pl.BlockSpec(shape, lambda i: (0, 0))
    return pl.pallas_call(
        _node_body,
        grid=(grid,),
        in_specs=[row, row, row, row, row, row, col, col,
                  full((1, 1)), full((13 * D, D)), full((1, D)), full((1, D)), full((1, D))],
        out_specs=row,
        out_shape=jax.ShapeDtypeStruct((NPAD, D), jnp.float32),
    )(h, bnd, s, q, m, n, lo, hi, mld, wp, b, g, be)


def kernel(x, rel0, W0, b0, g0, be0, rel1, W1, b1, g1, be1,
           rel2, W2, b2, g2, be2, edge_index, edge_type):
    src = edge_index[0]
    dst = edge_index[1]
    order = jnp.argsort(dst)
    src_s = jnp.zeros((EPAD,), jnp.int32).at[:E].set(src[order])
    dst_s = jnp.full((EPAD,), NPAD, jnp.int32).at[:E].set(dst[order])
    et_s = jnp.zeros((EPAD,), jnp.int32).at[:E].set(edge_type[order])
    rowptr = jnp.searchsorted(dst_s[:E], jnp.arange(N + 1, dtype=jnp.int32),
                              side="left").astype(jnp.int32)
    bounds = rowptr[jnp.minimum(jnp.arange(VT + 1) * NPT, N)]
    bounds = jnp.zeros((VT, 16), jnp.int32).at[:, 0].set(bounds[:VT]).at[:, 1].set(bounds[1:])

    lo = jnp.concatenate([rowptr[:N], jnp.zeros((NPAD - N,), jnp.int32)])
    hi = jnp.concatenate([rowptr[1:N + 1], jnp.zeros((NPAD - N,), jnp.int32)])
    mld = pl.pallas_call(
        _mld_body,
        out_shape=jax.ShapeDtypeStruct((1, 1), jnp.float32),
    )(lo.reshape(80, 128), hi.reshape(80, 128))
    lo2 = lo.reshape(NPAD, 1)
    hi2 = hi.reshape(NPAD, 1)

    xpad = jnp.zeros((NPAD, D), jnp.float32).at[:N].set(x)
    b2d = lambda a: a.reshape(1, D)

    def perm_w(W):
        return jnp.concatenate(
            [W[:D], W[D:].reshape(D, 4, 3, D).transpose(1, 2, 0, 3).reshape(12 * D, D)],
            axis=0)

    h = xpad
    for (rel, W, b, g, be) in ((rel0, W0, b0, g0, be0),
                               (rel1, W1, b1, g1, be1),
                               (rel2, W2, b2, g2, be2)):
        osum, osq, omx, omn = _edge_call(h, src_s, dst_s, et_s, bounds, rel)
        h = _node_call(h, xpad, osum, osq, omx, omn, lo2, hi2,
                       mld, perm_w(W), b2d(b), b2d(g), b2d(be))
    return h[:N]


# speedup vs baseline: 4.4811x; 1.7103x over previous
import jax
import jax.numpy as jnp
from jax import lax
from jax.experimental import pallas as pl
from jax.experimental.pallas import tpu as pltpu
from jax.experimental.pallas import tpu_sc as plsc

N = 10000
E = 320000
D = 128
NR = 36
EPS = 1e-5

VT = 128
NPT = 80
NPAD = VT * NPT
CHUNK = 128
SL = E // 32
SLP = 12032
E2 = 32 * SLP
BLK = 1024
FMAX = 3.4e38


def _bin_body(srch, dsth, eth,
              osrc, odst, oet, obnd,
              sch, dch, tch, cnt_v, cur_v, stg_s, stg_d, stg_t, loff_v):
    cid = lax.axis_index("c")
    sid = lax.axis_index("s")
    wid = sid * 2 + cid
    ebase = wid * SL
    iota = lax.iota(jnp.int32, 16)
    zero16 = jnp.zeros((16,), jnp.int32)

    def zc(i, _):
        o = pl.multiple_of(i * 16, 16)
        cnt_v[pl.ds(o, 16)] = zero16
        cur_v[pl.ds(o, 16)] = zero16
        return 0

    lax.fori_loop(0, VT // 16 + 1, zc, 0)

    def zs(i, _):
        off = pl.multiple_of(i * 16, 16)
        stg_s[pl.ds(off, 16)] = zero16
        stg_d[pl.ds(off, 16)] = zero16
        stg_t[pl.ds(off, 16)] = zero16
        return 0

    lax.fori_loop(0, SLP // 16, zs, 0)

    nch = (SL + CHUNK - 1) // CHUNK

    def countchunk(kc, _):
        base = ebase + kc * CHUNK
        pltpu.sync_copy(dsth.at[pl.ds(base, CHUNK)], dch.at[pl.ds(0, CHUNK)])
        hi = jnp.minimum(SL - kc * CHUNK, CHUNK)

        def ce(off, _):
            d = dch[pl.ds(off, 16)][0]
            b = ((d >> 4) * 6554) >> 15
            g16 = pl.multiple_of((b >> 4) * 16, 16)
            row = cnt_v[pl.ds(g16, 16)]
            hit = iota == jnp.broadcast_to(b & 15, (16,))
            cnt_v[pl.ds(g16, 16)] = row + jnp.where(hit, 1, 0)
            return 0

        lax.fori_loop(0, hi, ce, 0)
        return 0

    lax.fori_loop(0, nch, countchunk, 0)

    def scan_b(bb, carry):
        cval = cnt_v[pl.ds(bb, 16)][0]
        astart = carry
        aend = astart + cval
        r = pl.multiple_of((bb >> 3) * 16, 16)
        l0 = (2 * bb) & 15
        row = loff_v[pl.ds(r, 16)]
        row = jnp.where(iota == jnp.broadcast_to(l0, (16,)),
                        jnp.broadcast_to(astart, (16,)), row)
        row = jnp.where(iota == jnp.broadcast_to(l0 + 1, (16,)),
                        jnp.broadcast_to(aend, (16,)), row)
        loff_v[pl.ds(r, 16)] = row
        rc = pl.multiple_of((bb >> 4) * 16, 16)
        rowc = cur_v[pl.ds(rc, 16)]
        rowc = jnp.where(iota == jnp.broadcast_to(bb & 15, (16,)),
                         jnp.broadcast_to(astart, (16,)), rowc)
        cur_v[pl.ds(rc, 16)] = rowc
        return astart + (((cval + 15) >> 4) << 4)

    lax.fori_loop(0, VT, scan_b, jnp.int32(0))
    pltpu.sync_copy(loff_v, obnd.at[pl.ds(wid * 256, 256)])

    def scatchunk(kc, _):
        base = ebase + kc * CHUNK
        pltpu.sync_copy(srch.at[pl.ds(base, CHUNK)], sch.at[pl.ds(0, CHUNK)])
        pltpu.sync_copy(dsth.at[pl.ds(base, CHUNK)], dch.at[pl.ds(0, CHUNK)])
        pltpu.sync_copy(eth.at[pl.ds(base, CHUNK)], tch.at[pl.ds(0, CHUNK)])
        hi = jnp.minimum(SL - kc * CHUNK, CHUNK)

        def se(off, _):
            s = sch[pl.ds(off, 16)][0]
            d = dch[pl.ds(off, 16)][0]
            t = tch[pl.ds(off, 16)][0]
            b = ((d >> 4) * 6554) >> 15
            pos = cur_v[pl.ds(b, 16)][0]
            lane = pos & 15
            base16 = pl.multiple_of(pos - lane, 16)
            hit = iota == jnp.broadcast_to(lane, (16,))
            rs = stg_s[pl.ds(base16, 16)]
            stg_s[pl.ds(base16, 16)] = jnp.where(hit, jnp.broadcast_to(s, (16,)), rs)
            rd = stg_d[pl.ds(base16, 16)]
            stg_d[pl.ds(base16, 16)] = jnp.where(hit, jnp.broadcast_to(d, (16,)), rd)
            rt = stg_t[pl.ds(base16, 16)]
            stg_t[pl.ds(base16, 16)] = jnp.where(hit, jnp.broadcast_to(t, (16,)), rt)
            rc16 = pl.multiple_of((b >> 4) * 16, 16)
            rowc = cur_v[pl.ds(rc16, 16)]
            hitb = iota == jnp.broadcast_to(b & 15, (16,))
            cur_v[pl.ds(rc16, 16)] = rowc + jnp.where(hitb, 1, 0)
            return 0

        lax.fori_loop(0, hi, se, 0)
        return 0

    lax.fori_loop(0, nch, scatchunk, 0)

    pltpu.sync_copy(stg_s, osrc.at[pl.ds(wid * SLP, SLP)])
    pltpu.sync_copy(stg_d, odst.at[pl.ds(wid * SLP, SLP)])
    pltpu.sync_copy(stg_t, oet.at[pl.ds(wid * SLP, SLP)])


_bin_call = pl.kernel(
    _bin_body,
    out_type=[
        jax.ShapeDtypeStruct((E2,), jnp.int32),
        jax.ShapeDtypeStruct((E2,), jnp.int32),
        jax.ShapeDtypeStruct((E2,), jnp.int32),
        jax.ShapeDtypeStruct((8192,), jnp.int32),
    ],
    mesh=plsc.VectorSubcoreMesh(core_axis_name="c", subcore_axis_name="s"),
    scratch_types=[
        pltpu.VMEM((CHUNK + 16,), jnp.int32),
        pltpu.VMEM((CHUNK + 16,), jnp.int32),
        pltpu.VMEM((CHUNK + 16,), jnp.int32),
        pltpu.VMEM((VT + 16,), jnp.int32),
        pltpu.VMEM((VT + 16,), jnp.int32),
        pltpu.VMEM((SLP,), jnp.int32),
        pltpu.VMEM((SLP,), jnp.int32),
        pltpu.VMEM((SLP,), jnp.int32),
        pltpu.VMEM((256,), jnp.int32),
    ],
)


def _edge_body(xh, srch, dsth, eth, bndh, relh,
               osum, osq, omx, omn, ocnt,
               rel_v, bnd_v, idx_v, dst_v, et_v, rows_v,
               acc_s, acc_q, acc_m, acc_n, acc_c, sem):
    cid = lax.axis_index("c")
    sid = lax.axis_index("s")
    wid = sid * 2 + cid
    pltpu.sync_copy(relh, rel_v)
    pltpu.sync_copy(bndh, bnd_v.at[pl.ds(0, 8192)])
    zero = jnp.zeros((16,), jnp.float32)
    neg = jnp.full((16,), -FMAX, jnp.float32)
    pos = jnp.full((16,), FMAX, jnp.float32)
    onef0 = jnp.where(lax.iota(jnp.int32, 16) == 0, 1.0, 0.0).astype(jnp.float32)

    for p in range(4):
        v = wid * 4 + p
        node_base = v * NPT

        def init_row(i, _):
            for j in range(8):
                sl = pl.ds(j * 16, 16)
                acc_s[i, sl] = zero
                acc_q[i, sl] = zero
                acc_m[i, sl] = neg
                acc_n[i, sl] = pos
            acc_c[i, pl.ds(0, 16)] = zero
            return 0

        lax.fori_loop(0, NPT + 1, init_row, 0)

        def writer(w, _):
            bv = bnd_v[pl.ds(w * 256 + 2 * v, 16)]
            wbase = w * SLP
            e_start = wbase + bv[0]
            e_end = wbase + bv[1]
            c0 = (e_start // 8) * 8
            nch = (e_end - c0 + CHUNK - 1) // CHUNK

            def chunk(kc, _):
                base = c0 + kc * CHUNK
                pltpu.sync_copy(srch.at[pl.ds(base, CHUNK)], idx_v)
                pltpu.sync_copy(dsth.at[pl.ds(base, CHUNK)], dst_v.at[pl.ds(0, CHUNK)])
                pltpu.sync_copy(eth.at[pl.ds(base, CHUNK)], et_v.at[pl.ds(0, CHUNK)])
                pltpu.async_copy(xh.at[idx_v], rows_v, sem).wait()
                lo = jnp.maximum(e_start, base)
                hi = jnp.minimum(e_end, base + CHUNK)

                def edge(e, _):
                    off = e - base
                    ld = dst_v[pl.ds(off, 16)][0] - node_base
                    etk = et_v[pl.ds(off, 16)][0]
                    plsc.addupdate(acc_c.at[ld, pl.ds(0, 16)], onef0)
                    for j in range(8):
                        sl = pl.ds(j * 16, 16)
                        m = rel_v[etk, sl] * rows_v[off, sl]
                        plsc.addupdate(acc_s.at[ld, sl], m)
                        plsc.addupdate(acc_q.at[ld, sl], m * m)
                        acc_m[ld, sl] = jnp.maximum(acc_m[ld, sl], m)
                        acc_n[ld, sl] = jnp.minimum(acc_n[ld, sl], m)
                    return 0

                lax.fori_loop(lo, hi, edge, 0)
                return 0

            lax.fori_loop(0, nch, chunk, 0)
            return 0

        lax.fori_loop(0, 32, writer, 0)

        pltpu.sync_copy(acc_s.at[pl.ds(0, NPT)], osum.at[pl.ds(node_base, NPT)])
        pltpu.sync_copy(acc_q.at[pl.ds(0, NPT)], osq.at[pl.ds(node_base, NPT)])
        pltpu.sync_copy(acc_m.at[pl.ds(0, NPT)], omx.at[pl.ds(node_base, NPT)])
        pltpu.sync_copy(acc_n.at[pl.ds(0, NPT)], omn.at[pl.ds(node_base, NPT)])
        pltpu.sync_copy(acc_c.at[pl.ds(0, NPT)], ocnt.at[pl.ds(node_base, NPT)])


_edge_call = pl.kernel(
    _edge_body,
    out_type=[jax.ShapeDtypeStruct((NPAD, D), jnp.float32)] * 4
    + [jax.ShapeDtypeStruct((NPAD, 16), jnp.float32)],
    mesh=plsc.VectorSubcoreMesh(core_axis_name="c", subcore_axis_name="s"),
    scratch_types=[
        pltpu.VMEM((NR, D), jnp.float32),
        pltpu.VMEM((8192 + 16,), jnp.int32),
        pltpu.VMEM((CHUNK,), jnp.int32),
        pltpu.VMEM((CHUNK + 16,), jnp.int32),
        pltpu.VMEM((CHUNK + 16,), jnp.int32),
        pltpu.VMEM((CHUNK, D), jnp.float32),
        pltpu.VMEM((NPT + 1, D), jnp.float32),
        pltpu.VMEM((NPT + 1, D), jnp.float32),
        pltpu.VMEM((NPT + 1, D), jnp.float32),
        pltpu.VMEM((NPT + 1, D), jnp.float32),
        pltpu.VMEM((NPT + 1, 16), jnp.float32),
        pltpu.SemaphoreType.DMA,
    ],
)


def _deg_body(cnt_ref, deg_ref, mld_ref):
    degf = cnt_ref[...] + 1.0
    deg_ref[...] = degf
    mld_ref[...] = (jnp.sum(jnp.log(degf)) / jnp.float32(N)).reshape(1, 1)


def _node_body(h_ref, bnd_ref, s_ref, q_ref, m_ref, n_ref,
               deg_ref, mld_ref, wp_ref, b_ref, g_ref, be_ref, out_ref):
    x = bnd_ref[...]
    h = h_ref[...]
    deg = deg_ref[...]
    st = s_ref[...] + x
    qt = q_ref[...] + x * x
    mx = jnp.maximum(m_ref[...], x)
    mn = jnp.minimum(n_ref[...], x)
    inv_deg = 1.0 / deg
    mean = st * inv_deg
    std = jnp.sqrt(jnp.clip(qt * inv_deg - mean * mean, 1e-10, None))
    sc = jnp.log(deg) / mld_ref[0, 0]
    inv = 1.0 / jnp.clip(sc, 1e-2, None)
    parts = [h]
    for f in (mean, mx, mn, std):
        parts += [f, f * sc, f * inv]
    feats = jnp.concatenate(parts, axis=1)
    out = lax.dot_general(feats, wp_ref[...], (((1,), (0,)), ((), ())),
                          preferred_element_type=jnp.float32) + b_ref[...]
    mu = jnp.mean(out, axis=-1, keepdims=True)
    var = jnp.mean((out - mu) ** 2, axis=-1, keepdims=True)
    out = (out - mu) / jnp.sqrt(var + EPS) * g_ref[...] + be_ref[...]
    out_ref[...] = jnp.maximum(out, 0.0) + h


def _node_call(h, bnd, s, q, m, n, deg, mld, wp, b, g, be):
    grid = NPAD // BLK
    row = pl.BlockSpec((BLK, D), lambda i: (i, 0))
    col = pl.BlockSpec((BLK, 1), lambda i: (i, 0))
    full = lambda shape: pl.BlockSpec(shape, lambda i: (0, 0))
    return pl.pallas_call(
        _node_body,
        grid=(grid,),
        in_specs=[row, row, row, row, row, row, col,
                  full((1, 1)), full((13 * D, D)), full((1, D)), full((1, D)), full((1, D))],
        out_specs=row,
        out_shape=jax.ShapeDtypeStruct((NPAD, D), jnp.float32),
    )(h, bnd, s, q, m, n, deg, mld, wp, b, g, be)


def kernel(x, rel0, W0, b0, g0, be0, rel1, W1, b1, g1, be1,
           rel2, W2, b2, g2, be2, edge_index, edge_type):
    pad = jnp.zeros((CHUNK,), jnp.int32)
    src = jnp.concatenate([edge_index[0], pad])
    dst = jnp.concatenate([edge_index[1], pad])
    et = jnp.concatenate([edge_type, pad])

    src_s, dst_s, et_s, bounds = _bin_call(src, dst, et)

    xpad = jnp.zeros((NPAD, D), jnp.float32).at[:N].set(x)
    b2d = lambda a: a.reshape(1, D)

    def perm_w(W):
        return jnp.concatenate(
            [W[:D], W[D:].reshape(D, 4, 3, D).transpose(1, 2, 0, 3).reshape(12 * D, D)],
            axis=0)

    h = xpad
    deg2 = None
    mld = None
    for (rel, W, b, g, be) in ((rel0, W0, b0, g0, be0),
                               (rel1, W1, b1, g1, be1),
                               (rel2, W2, b2, g2, be2)):
        osum, osq, omx, omn, ocnt = _edge_call(h, src_s, dst_s, et_s, bounds, rel)
        if deg2 is None:
            deg, mld = pl.pallas_call(
                _deg_body,
                out_shape=[jax.ShapeDtypeStruct((80, 128), jnp.float32),
                           jax.ShapeDtypeStruct((1, 1), jnp.float32)],
            )(ocnt[:, 0].reshape(80, 128))
            deg2 = deg.reshape(NPAD, 1)
        h = _node_call(h, xpad, osum, osq, omx, omn, deg2,
                       mld, perm_w(W), b2d(b), b2d(g), b2d(be))
    return h[:N]
